# Initial kernel scaffold; baseline (speedup 1.0000x reference)
import jax
import jax.numpy as jnp
from jax.experimental import pallas as pl


def kernel(nodes, edges, W_mlp, W_root, b_root, gn_scale, gn_bias, senders, receivers):
    raise NotImplementedError("write your pallas kernel here")


# speedup vs baseline: 2.4677x; 2.4677x over previous
import functools

import jax
import jax.numpy as jnp
from jax import lax
from jax.experimental import pallas as pl
from jax.experimental.pallas import tpu as pltpu
from jax.experimental.pallas import tpu_sc as plsc

N = 10000
E = 160000
D = 48
DE = 16
EPS = 1e-6

DPAD = 64
GPAD = 128
NW = 32
CH = 128
E_PAD = 163840
CHUNKS_PER_W = E_PAD // (NW * CH)
TRASH = N
N_PAD = 10112
RPT = N_PAD // 16
BE = 2048
BN = 2000


def _sc_mesh():
    return plsc.VectorSubcoreMesh(core_axis_name="c", subcore_axis_name="s")


def _gather_rows(nodes, senders_p):
    @functools.partial(
        pl.kernel,
        mesh=_sc_mesh(),
        out_type=jax.ShapeDtypeStruct((E_PAD, GPAD), jnp.float32),
        scratch_types=[
            pltpu.VMEM((CH,), jnp.int32),
            pltpu.VMEM((CH, GPAD), jnp.float32),
            pltpu.SemaphoreType.DMA,
        ],
    )
    def k(nodes_hbm, snd_hbm, out_hbm, idx_v, rows_v, sem):
        wid = lax.axis_index("s") * 2 + lax.axis_index("c")
        base = wid * (CHUNKS_PER_W * CH)

        def body(i, carry):
            off = base + i * CH
            pltpu.sync_copy(snd_hbm.at[pl.ds(off, CH)], idx_v)
            pltpu.async_copy(nodes_hbm.at[idx_v], rows_v, sem).wait()
            pltpu.sync_copy(rows_v, out_hbm.at[pl.ds(off, CH)])
            return carry

        lax.fori_loop(0, CHUNKS_PER_W, body, 0)

    return k(nodes, senders_p)


def _edge_matmul(x_src, edges_p, w_cat):
    def body(x_ref, e_ref, w_ref, o_ref):
        x = x_ref[:, :D]
        e = e_ref[...]
        y = jnp.dot(x, w_ref[...], preferred_element_type=jnp.float32)
        acc = jnp.zeros((BE, DPAD), jnp.float32)
        for f in range(DE):
            acc = acc + e[:, f:f + 1] * y[:, f * DPAD:(f + 1) * DPAD]
        col = lax.broadcasted_iota(jnp.int32, (BE, DPAD), 1)
        o_ref[...] = acc + jnp.where(col == D, 1.0, 0.0)

    return pl.pallas_call(
        body,
        grid=(E_PAD // BE,),
        in_specs=[
            pl.BlockSpec((BE, GPAD), lambda i: (i, 0)),
            pl.BlockSpec((BE, DE), lambda i: (i, 0)),
            pl.BlockSpec((D, DE * DPAD), lambda i: (0, 0)),
        ],
        out_specs=pl.BlockSpec((BE, DPAD), lambda i: (i, 0)),
        out_shape=jax.ShapeDtypeStruct((E_PAD, DPAD), jnp.float32),
    )(x_src, edges_p, w_cat)


def _scatter_add(msgs, receivers_p, zinit):
    @functools.partial(
        pl.kernel,
        mesh=_sc_mesh(),
        out_type=jax.ShapeDtypeStruct((2, N_PAD, DPAD), jnp.float32),
        scratch_types=[
            pltpu.VMEM((CH,), jnp.int32),
            pltpu.VMEM((CH, DPAD), jnp.float32),
            pltpu.VMEM_SHARED((N_PAD, DPAD), jnp.float32),
        ],
    )
    def k(msgs_hbm, rcv_hbm, z_hbm, out_hbm, idx_v, m_v, agg_sh):
        c = lax.axis_index("c")
        s = lax.axis_index("s")
        wid = s * 2 + c
        pltpu.sync_copy(z_hbm.at[pl.ds(s * RPT, RPT)], agg_sh.at[pl.ds(s * RPT, RPT)])
        plsc.subcore_barrier()
        base = wid * (CHUNKS_PER_W * CH)

        def body(i, carry):
            off = base + i * CH
            pltpu.sync_copy(rcv_hbm.at[pl.ds(off, CH)], idx_v)
            pltpu.sync_copy(msgs_hbm.at[pl.ds(off, CH)], m_v)
            pltpu.sync_copy(m_v, agg_sh.at[idx_v], add=True)
            return carry

        lax.fori_loop(0, CHUNKS_PER_W, body, 0)
        plsc.subcore_barrier()
        pltpu.sync_copy(agg_sh.at[pl.ds(s * RPT, RPT)],
                        out_hbm.at[c, pl.ds(s * RPT, RPT)])

    return k(msgs, receivers_p, zinit)


def _finalize(p0, p1, nodes, w_root, b_root, gn_scale, gn_bias):
    def body(p0_ref, p1_ref, n_ref, wr_ref, br_ref, gs_ref, gb_ref, o_ref):
        agg = p0_ref[...] + p1_ref[...]
        deg = agg[:, D:D + 1]
        a = agg[:, :D] / jnp.maximum(deg, 1.0)
        h = a + jnp.dot(n_ref[...], wr_ref[...],
                        preferred_element_type=jnp.float32) + br_ref[...]
        mean = h
        cen = h - mean
        var = cen * cen
        y = cen * lax.rsqrt(var + EPS)
        o_ref[...] = y * gs_ref[...] + gb_ref[...]

    return pl.pallas_call(
        body,
        grid=(N // BN,),
        in_specs=[
            pl.BlockSpec((BN, DPAD), lambda i: (i, 0)),
            pl.BlockSpec((BN, DPAD), lambda i: (i, 0)),
            pl.BlockSpec((BN, D), lambda i: (i, 0)),
            pl.BlockSpec((D, D), lambda i: (0, 0)),
            pl.BlockSpec((1, D), lambda i: (0, 0)),
            pl.BlockSpec((1, D), lambda i: (0, 0)),
            pl.BlockSpec((1, D), lambda i: (0, 0)),
        ],
        out_specs=pl.BlockSpec((BN, D), lambda i: (i, 0)),
        out_shape=jax.ShapeDtypeStruct((N, D), jnp.float32),
    )(p0, p1, nodes, w_root, b_root, gn_scale, gn_bias)


def kernel(nodes, edges, W_mlp, W_root, b_root, gn_scale, gn_bias, senders, receivers):
    pad = E_PAD - E
    senders_p = jnp.concatenate([senders, jnp.zeros((pad,), jnp.int32)])
    receivers_p = jnp.concatenate([receivers, jnp.full((pad,), TRASH, jnp.int32)])
    edges_p = jnp.pad(edges, ((0, pad), (0, 0)))
    w_cat = jnp.pad(W_mlp, ((0, 0), (0, 0), (0, DPAD - D))).transpose(1, 0, 2)
    w_cat = w_cat.reshape(D, DE * DPAD)
    zinit = jnp.zeros((N_PAD, DPAD), jnp.float32)

    nodes_g = jnp.pad(nodes, ((0, 0), (0, GPAD - D)))
    x_src = _gather_rows(nodes_g, senders_p)
    msgs = _edge_matmul(x_src, edges_p, w_cat)
    partials = _scatter_add(msgs, receivers_p, zinit)
    out = _finalize(partials[0, :N], partials[1, :N], nodes,
                    W_root, b_root.reshape(1, D),
                    gn_scale.reshape(1, D), gn_bias.reshape(1, D))
    return out


# speedup vs baseline: 2.7523x; 1.1153x over previous
import functools

import jax
import jax.numpy as jnp
from jax import lax
from jax.experimental import pallas as pl
from jax.experimental.pallas import tpu as pltpu
from jax.experimental.pallas import tpu_sc as plsc

N = 10000
E = 160000
D = 48
DE = 16
EPS = 1e-6

DPAD = 64
GPAD = 128
NW = 32
CH = 128
E_PAD = 163840
CHUNKS_PER_W = E_PAD // (NW * CH)
TRASH = N
N_PAD = 10112
RPT = N_PAD // 16
BE = 2048
BN = 2000


def _sc_mesh():
    return plsc.VectorSubcoreMesh(core_axis_name="c", subcore_axis_name="s")


def _gather_rows(nodes, senders_p):
    @functools.partial(
        pl.kernel,
        mesh=_sc_mesh(),
        out_type=jax.ShapeDtypeStruct((E_PAD, D), jnp.float32),
        scratch_types=[
            pltpu.VMEM((CH,), jnp.int32),
            pltpu.VMEM((CH, D), jnp.float32),
            pltpu.SemaphoreType.DMA,
        ],
        compiler_params=pltpu.CompilerParams(use_tc_tiling_on_sc=False),
    )
    def k(nodes_hbm, snd_hbm, out_hbm, idx_v, rows_v, sem):
        wid = lax.axis_index("s") * 2 + lax.axis_index("c")
        base = wid * (CHUNKS_PER_W * CH)

        def body(i, carry):
            off = base + i * CH
            pltpu.sync_copy(snd_hbm.at[pl.ds(off, CH)], idx_v)
            pltpu.async_copy(nodes_hbm.at[idx_v], rows_v, sem).wait()
            pltpu.sync_copy(rows_v, out_hbm.at[pl.ds(off, CH)])
            return carry

        lax.fori_loop(0, CHUNKS_PER_W, body, 0)

    return k(nodes, senders_p)


def _edge_matmul(x_src, edges_p, w_cat):
    def body(x_ref, e_ref, w_ref, o_ref):
        x = x_ref[:, :D]
        e = e_ref[...]
        y = jnp.dot(x, w_ref[...], preferred_element_type=jnp.float32)
        acc = jnp.zeros((BE, DPAD), jnp.float32)
        for f in range(DE):
            acc = acc + e[:, f:f + 1] * y[:, f * DPAD:(f + 1) * DPAD]
        col = lax.broadcasted_iota(jnp.int32, (BE, DPAD), 1)
        o_ref[...] = acc + jnp.where(col == D, 1.0, 0.0)

    return pl.pallas_call(
        body,
        grid=(E_PAD // BE,),
        in_specs=[
            pl.BlockSpec((BE, D), lambda i: (i, 0)),
            pl.BlockSpec((BE, DE), lambda i: (i, 0)),
            pl.BlockSpec((D, DE * DPAD), lambda i: (0, 0)),
        ],
        out_specs=pl.BlockSpec((BE, DPAD), lambda i: (i, 0)),
        out_shape=jax.ShapeDtypeStruct((E_PAD, DPAD), jnp.float32),
    )(x_src, edges_p, w_cat)


def _scatter_add(msgs, receivers_p, zinit):
    @functools.partial(
        pl.kernel,
        mesh=_sc_mesh(),
        out_type=jax.ShapeDtypeStruct((2, N_PAD, DPAD), jnp.float32),
        scratch_types=[
            pltpu.VMEM((CH,), jnp.int32),
            pltpu.VMEM((CH, DPAD), jnp.float32),
            pltpu.VMEM_SHARED((N_PAD, DPAD), jnp.float32),
        ],
    )
    def k(msgs_hbm, rcv_hbm, z_hbm, out_hbm, idx_v, m_v, agg_sh):
        c = lax.axis_index("c")
        s = lax.axis_index("s")
        wid = s * 2 + c
        pltpu.sync_copy(z_hbm.at[pl.ds(s * RPT, RPT)], agg_sh.at[pl.ds(s * RPT, RPT)])
        plsc.subcore_barrier()
        base = wid * (CHUNKS_PER_W * CH)

        def body(i, carry):
            off = base + i * CH
            pltpu.sync_copy(rcv_hbm.at[pl.ds(off, CH)], idx_v)
            pltpu.sync_copy(msgs_hbm.at[pl.ds(off, CH)], m_v)
            pltpu.sync_copy(m_v, agg_sh.at[idx_v], add=True)
            return carry

        lax.fori_loop(0, CHUNKS_PER_W, body, 0)
        plsc.subcore_barrier()
        pltpu.sync_copy(agg_sh.at[pl.ds(s * RPT, RPT)],
                        out_hbm.at[c, pl.ds(s * RPT, RPT)])

    return k(msgs, receivers_p, zinit)


def _finalize(p0, p1, nodes, w_root, b_root, gn_scale, gn_bias):
    def body(p0_ref, p1_ref, n_ref, wr_ref, br_ref, gs_ref, gb_ref, o_ref):
        agg = p0_ref[...] + p1_ref[...]
        deg = agg[:, D:D + 1]
        a = agg[:, :D] / jnp.maximum(deg, 1.0)
        h = a + jnp.dot(n_ref[...], wr_ref[...],
                        preferred_element_type=jnp.float32) + br_ref[...]
        mean = h
        cen = h - mean
        var = cen * cen
        y = cen * lax.rsqrt(var + EPS)
        o_ref[...] = y * gs_ref[...] + gb_ref[...]

    return pl.pallas_call(
        body,
        grid=(N // BN,),
        in_specs=[
            pl.BlockSpec((BN, DPAD), lambda i: (i, 0)),
            pl.BlockSpec((BN, DPAD), lambda i: (i, 0)),
            pl.BlockSpec((BN, D), lambda i: (i, 0)),
            pl.BlockSpec((D, D), lambda i: (0, 0)),
            pl.BlockSpec((1, D), lambda i: (0, 0)),
            pl.BlockSpec((1, D), lambda i: (0, 0)),
            pl.BlockSpec((1, D), lambda i: (0, 0)),
        ],
        out_specs=pl.BlockSpec((BN, D), lambda i: (i, 0)),
        out_shape=jax.ShapeDtypeStruct((N, D), jnp.float32),
    )(p0, p1, nodes, w_root, b_root, gn_scale, gn_bias)


def kernel(nodes, edges, W_mlp, W_root, b_root, gn_scale, gn_bias, senders, receivers):
    pad = E_PAD - E
    senders_p = jnp.concatenate([senders, jnp.zeros((pad,), jnp.int32)])
    receivers_p = jnp.concatenate([receivers, jnp.full((pad,), TRASH, jnp.int32)])
    edges_p = jnp.pad(edges, ((0, pad), (0, 0)))
    w_cat = jnp.pad(W_mlp, ((0, 0), (0, 0), (0, DPAD - D))).transpose(1, 0, 2)
    w_cat = w_cat.reshape(D, DE * DPAD)
    zinit = jnp.zeros((N_PAD, DPAD), jnp.float32)

    x_src = _gather_rows(nodes, senders_p)
    msgs = _edge_matmul(x_src, edges_p, w_cat)
    partials = _scatter_add(msgs, receivers_p, zinit)
    out = _finalize(partials[0, :N], partials[1, :N], nodes,
                    W_root, b_root.reshape(1, D),
                    gn_scale.reshape(1, D), gn_bias.reshape(1, D))
    return out


# speedup vs baseline: 2.8518x; 1.0362x over previous
import functools

import jax
import jax.numpy as jnp
from jax import lax
from jax.experimental import pallas as pl
from jax.experimental.pallas import tpu as pltpu
from jax.experimental.pallas import tpu_sc as plsc

N = 10000
E = 160000
D = 48
DE = 16
EPS = 1e-6

DPAD = 64
GPAD = 128
NW = 32
CH = 128
E_PAD = 163840
CHUNKS_PER_W = E_PAD // (NW * CH)
TRASH = N
N_PAD = 10112
RPT = N_PAD // 16
BE = 2048
BN = 2000


def _sc_mesh():
    return plsc.VectorSubcoreMesh(core_axis_name="c", subcore_axis_name="s")


def _gather_rows(nodes, senders_p):
    nbuf = 4

    @functools.partial(
        pl.kernel,
        mesh=_sc_mesh(),
        out_type=jax.ShapeDtypeStruct((E_PAD, D), jnp.float32),
        scratch_types=[
            pltpu.VMEM((CHUNKS_PER_W * CH,), jnp.int32),
            pltpu.VMEM((nbuf, CH, D), jnp.float32),
            [pltpu.SemaphoreType.DMA] * nbuf,
            [pltpu.SemaphoreType.DMA] * nbuf,
        ],
        compiler_params=pltpu.CompilerParams(use_tc_tiling_on_sc=False),
    )
    def k(nodes_hbm, snd_hbm, out_hbm, idx_v, rows_v, gsems, wsems):
        wid = lax.axis_index("s") * 2 + lax.axis_index("c")
        base = wid * (CHUNKS_PER_W * CH)
        pltpu.sync_copy(snd_hbm.at[pl.ds(base, CHUNKS_PER_W * CH)], idx_v)

        def body(j, carry):
            gh = []
            for b in range(nbuf):
                i = j * nbuf + b
                gh.append(pltpu.async_copy(
                    nodes_hbm.at[idx_v.at[pl.ds(i * CH, CH)]],
                    rows_v.at[b], gsems[b]))
            wh = []
            for b in range(nbuf):
                i = j * nbuf + b
                gh[b].wait()
                wh.append(pltpu.async_copy(
                    rows_v.at[b], out_hbm.at[pl.ds(base + i * CH, CH)],
                    wsems[b]))
            for b in range(nbuf):
                wh[b].wait()
            return carry

        lax.fori_loop(0, CHUNKS_PER_W // nbuf, body, 0)

    return k(nodes, senders_p)


def _edge_matmul(x_src, edges_p, w_cat):
    def body(x_ref, e_ref, w_ref, o_ref):
        x = x_ref[:, :D]
        e = e_ref[...]
        y = jnp.dot(x, w_ref[...], preferred_element_type=jnp.float32)
        acc = jnp.zeros((BE, DPAD), jnp.float32)
        for f in range(DE):
            acc = acc + e[:, f:f + 1] * y[:, f * DPAD:(f + 1) * DPAD]
        col = lax.broadcasted_iota(jnp.int32, (BE, DPAD), 1)
        o_ref[...] = acc + jnp.where(col == D, 1.0, 0.0)

    return pl.pallas_call(
        body,
        grid=(E_PAD // BE,),
        in_specs=[
            pl.BlockSpec((BE, D), lambda i: (i, 0)),
            pl.BlockSpec((BE, DE), lambda i: (i, 0)),
            pl.BlockSpec((D, DE * DPAD), lambda i: (0, 0)),
        ],
        out_specs=pl.BlockSpec((BE, DPAD), lambda i: (i, 0)),
        out_shape=jax.ShapeDtypeStruct((E_PAD, DPAD), jnp.float32),
    )(x_src, edges_p, w_cat)


def _scatter_add(msgs, receivers_p, zinit):
    @functools.partial(
        pl.kernel,
        mesh=_sc_mesh(),
        out_type=jax.ShapeDtypeStruct((2, N_PAD, DPAD), jnp.float32),
        scratch_types=[
            pltpu.VMEM((CH,), jnp.int32),
            pltpu.VMEM((CH, DPAD), jnp.float32),
            pltpu.VMEM_SHARED((N_PAD, DPAD), jnp.float32),
        ],
    )
    def k(msgs_hbm, rcv_hbm, z_hbm, out_hbm, idx_v, m_v, agg_sh):
        c = lax.axis_index("c")
        s = lax.axis_index("s")
        wid = s * 2 + c
        pltpu.sync_copy(z_hbm.at[pl.ds(s * RPT, RPT)], agg_sh.at[pl.ds(s * RPT, RPT)])
        plsc.subcore_barrier()
        base = wid * (CHUNKS_PER_W * CH)

        def body(i, carry):
            off = base + i * CH
            pltpu.sync_copy(rcv_hbm.at[pl.ds(off, CH)], idx_v)
            pltpu.sync_copy(msgs_hbm.at[pl.ds(off, CH)], m_v)
            pltpu.sync_copy(m_v, agg_sh.at[idx_v], add=True)
            return carry

        lax.fori_loop(0, CHUNKS_PER_W, body, 0)
        plsc.subcore_barrier()
        pltpu.sync_copy(agg_sh.at[pl.ds(s * RPT, RPT)],
                        out_hbm.at[c, pl.ds(s * RPT, RPT)])

    return k(msgs, receivers_p, zinit)


def _finalize(p0, p1, nodes, w_root, b_root, gn_scale, gn_bias):
    def body(p0_ref, p1_ref, n_ref, wr_ref, br_ref, gs_ref, gb_ref, o_ref):
        agg = p0_ref[...] + p1_ref[...]
        deg = agg[:, D:D + 1]
        a = agg[:, :D] / jnp.maximum(deg, 1.0)
        h = a + jnp.dot(n_ref[...], wr_ref[...],
                        preferred_element_type=jnp.float32) + br_ref[...]
        mean = h
        cen = h - mean
        var = cen * cen
        y = cen * lax.rsqrt(var + EPS)
        o_ref[...] = y * gs_ref[...] + gb_ref[...]

    return pl.pallas_call(
        body,
        grid=(N // BN,),
        in_specs=[
            pl.BlockSpec((BN, DPAD), lambda i: (i, 0)),
            pl.BlockSpec((BN, DPAD), lambda i: (i, 0)),
            pl.BlockSpec((BN, D), lambda i: (i, 0)),
            pl.BlockSpec((D, D), lambda i: (0, 0)),
            pl.BlockSpec((1, D), lambda i: (0, 0)),
            pl.BlockSpec((1, D), lambda i: (0, 0)),
            pl.BlockSpec((1, D), lambda i: (0, 0)),
        ],
        out_specs=pl.BlockSpec((BN, D), lambda i: (i, 0)),
        out_shape=jax.ShapeDtypeStruct((N, D), jnp.float32),
    )(p0, p1, nodes, w_root, b_root, gn_scale, gn_bias)


def kernel(nodes, edges, W_mlp, W_root, b_root, gn_scale, gn_bias, senders, receivers):
    pad = E_PAD - E
    senders_p = jnp.concatenate([senders, jnp.zeros((pad,), jnp.int32)])
    receivers_p = jnp.concatenate([receivers, jnp.full((pad,), TRASH, jnp.int32)])
    edges_p = jnp.pad(edges, ((0, pad), (0, 0)))
    w_cat = jnp.pad(W_mlp, ((0, 0), (0, 0), (0, DPAD - D))).transpose(1, 0, 2)
    w_cat = w_cat.reshape(D, DE * DPAD)
    zinit = jnp.zeros((N_PAD, DPAD), jnp.float32)

    x_src = _gather_rows(nodes, senders_p)
    msgs = _edge_matmul(x_src, edges_p, w_cat)
    partials = _scatter_add(msgs, receivers_p, zinit)
    out = _finalize(partials[0, :N], partials[1, :N], nodes,
                    W_root, b_root.reshape(1, D),
                    gn_scale.reshape(1, D), gn_bias.reshape(1, D))
    return out


# speedup vs baseline: 3.1865x; 1.1173x over previous
import functools

import jax
import jax.numpy as jnp
from jax import lax
from jax.experimental import pallas as pl
from jax.experimental.pallas import tpu as pltpu
from jax.experimental.pallas import tpu_sc as plsc

N = 10000
E = 160000
D = 48
DE = 16
EPS = 1e-6

DPAD = 64
NW = 32
CH = 128
E_PAD = 163840
NHALF = 2
EH = E_PAD // NHALF
CPW = EH // (NW * CH)
TRASH = N
N_PAD = 10112
RPT = N_PAD // 16
BE = 2048
BN = 2000


def _sc_mesh():
    return plsc.VectorSubcoreMesh(core_axis_name="c", subcore_axis_name="s")


def _gather_rows(nodes, senders_h):
    nbuf = 4

    @functools.partial(
        pl.kernel,
        mesh=_sc_mesh(),
        out_type=jax.ShapeDtypeStruct((EH, D), jnp.float32),
        scratch_types=[
            pltpu.VMEM((CPW * CH,), jnp.int32),
            pltpu.VMEM((nbuf, CH, D), jnp.float32),
            [pltpu.SemaphoreType.DMA] * nbuf,
            [pltpu.SemaphoreType.DMA] * nbuf,
        ],
        compiler_params=pltpu.CompilerParams(use_tc_tiling_on_sc=False),
    )
    def k(nodes_hbm, snd_hbm, out_hbm, idx_v, rows_v, gsems, wsems):
        wid = lax.axis_index("s") * 2 + lax.axis_index("c")
        base = wid * (CPW * CH)
        pltpu.sync_copy(snd_hbm.at[pl.ds(base, CPW * CH)], idx_v)

        def body(j, carry):
            gh = []
            for b in range(nbuf):
                i = j * nbuf + b
                gh.append(pltpu.async_copy(
                    nodes_hbm.at[idx_v.at[pl.ds(i * CH, CH)]],
                    rows_v.at[b], gsems[b]))
            wh = []
            for b in range(nbuf):
                i = j * nbuf + b
                gh[b].wait()
                wh.append(pltpu.async_copy(
                    rows_v.at[b], out_hbm.at[pl.ds(base + i * CH, CH)],
                    wsems[b]))
            for b in range(nbuf):
                wh[b].wait()
            return carry

        lax.fori_loop(0, CPW // nbuf, body, 0)

    return k(nodes, senders_h)


def _edge_matmul(x_src, edges_h, w_cat):
    def body(x_ref, e_ref, w_ref, o_ref):
        x = x_ref[...]
        e = e_ref[...]
        y = jnp.dot(x, w_ref[...], preferred_element_type=jnp.float32)
        acc = jnp.zeros((BE, DPAD), jnp.float32)
        for f in range(DE):
            acc = acc + e[:, f:f + 1] * y[:, f * DPAD:(f + 1) * DPAD]
        col = lax.broadcasted_iota(jnp.int32, (BE, DPAD), 1)
        o_ref[...] = acc + jnp.where(col == D, 1.0, 0.0)

    return pl.pallas_call(
        body,
        grid=(EH // BE,),
        in_specs=[
            pl.BlockSpec((BE, D), lambda i: (i, 0)),
            pl.BlockSpec((BE, DE), lambda i: (i, 0)),
            pl.BlockSpec((D, DE * DPAD), lambda i: (0, 0)),
        ],
        out_specs=pl.BlockSpec((BE, DPAD), lambda i: (i, 0)),
        out_shape=jax.ShapeDtypeStruct((EH, DPAD), jnp.float32),
    )(x_src, edges_h, w_cat)


def _scatter_add(msgs, receivers_h, zinit):
    @functools.partial(
        pl.kernel,
        mesh=_sc_mesh(),
        out_type=jax.ShapeDtypeStruct((2, N_PAD, DPAD), jnp.float32),
        scratch_types=[
            pltpu.VMEM((CH,), jnp.int32),
            pltpu.VMEM((CH, DPAD), jnp.float32),
            pltpu.VMEM_SHARED((N_PAD, DPAD), jnp.float32),
        ],
    )
    def k(msgs_hbm, rcv_hbm, z_hbm, out_hbm, idx_v, m_v, agg_sh):
        c = lax.axis_index("c")
        s = lax.axis_index("s")
        wid = s * 2 + c
        pltpu.sync_copy(z_hbm.at[pl.ds(s * RPT, RPT)], agg_sh.at[pl.ds(s * RPT, RPT)])
        plsc.subcore_barrier()
        base = wid * (CPW * CH)

        def body(i, carry):
            off = base + i * CH
            pltpu.sync_copy(rcv_hbm.at[pl.ds(off, CH)], idx_v)
            pltpu.sync_copy(msgs_hbm.at[pl.ds(off, CH)], m_v)
            pltpu.sync_copy(m_v, agg_sh.at[idx_v], add=True)
            return carry

        lax.fori_loop(0, CPW, body, 0)
        plsc.subcore_barrier()
        pltpu.sync_copy(agg_sh.at[pl.ds(s * RPT, RPT)],
                        out_hbm.at[c, pl.ds(s * RPT, RPT)])

    return k(msgs, receivers_h, zinit)


def _finalize(pa, pb, nodes, w_root, b_root, gn_scale, gn_bias):
    def body(pa_ref, pb_ref, n_ref, wr_ref, br_ref, gs_ref, gb_ref, o_ref):
        agg = pa_ref[0] + pa_ref[1] + pb_ref[0] + pb_ref[1]
        deg = agg[:, D:D + 1]
        a = agg[:, :D] / jnp.maximum(deg, 1.0)
        h = a + jnp.dot(n_ref[...], wr_ref[...],
                        preferred_element_type=jnp.float32) + br_ref[...]
        mean = h
        cen = h - mean
        var = cen * cen
        y = cen * lax.rsqrt(var + EPS)
        o_ref[...] = y * gs_ref[...] + gb_ref[...]

    return pl.pallas_call(
        body,
        grid=(N // BN,),
        in_specs=[
            pl.BlockSpec((2, BN, DPAD), lambda i: (0, i, 0)),
            pl.BlockSpec((2, BN, DPAD), lambda i: (0, i, 0)),
            pl.BlockSpec((BN, D), lambda i: (i, 0)),
            pl.BlockSpec((D, D), lambda i: (0, 0)),
            pl.BlockSpec((1, D), lambda i: (0, 0)),
            pl.BlockSpec((1, D), lambda i: (0, 0)),
            pl.BlockSpec((1, D), lambda i: (0, 0)),
        ],
        out_specs=pl.BlockSpec((BN, D), lambda i: (i, 0)),
        out_shape=jax.ShapeDtypeStruct((N, D), jnp.float32),
    )(pa, pb, nodes, w_root, b_root, gn_scale, gn_bias)


def kernel(nodes, edges, W_mlp, W_root, b_root, gn_scale, gn_bias, senders, receivers):
    pad = E_PAD - E
    senders_p = jnp.concatenate([senders, jnp.zeros((pad,), jnp.int32)])
    receivers_p = jnp.concatenate([receivers, jnp.full((pad,), TRASH, jnp.int32)])
    edges_p = jnp.pad(edges, ((0, pad), (0, 0)))
    w_cat = jnp.pad(W_mlp, ((0, 0), (0, 0), (0, DPAD - D))).transpose(1, 0, 2)
    w_cat = w_cat.reshape(D, DE * DPAD)
    zinit = jnp.zeros((N_PAD, DPAD), jnp.float32)

    parts = []
    for h in range(NHALF):
        sl = slice(h * EH, (h + 1) * EH)
        x_h = _gather_rows(nodes, senders_p[sl])
        m_h = _edge_matmul(x_h, edges_p[sl], w_cat)
        parts.append(_scatter_add(m_h, receivers_p[sl], zinit))
    out = _finalize(parts[0], parts[1], nodes,
                    W_root, b_root.reshape(1, D),
                    gn_scale.reshape(1, D), gn_bias.reshape(1, D))
    return out


# speedup vs baseline: 3.2739x; 1.0274x over previous
import functools

import jax
import jax.numpy as jnp
from jax import lax
from jax.experimental import pallas as pl
from jax.experimental.pallas import tpu as pltpu
from jax.experimental.pallas import tpu_sc as plsc

N = 10000
E = 160000
D = 48
DE = 16
EPS = 1e-6

DPAD = 64
NW = 32
CH = 128
E_PAD = 163840
NHALF = 4
EH = E_PAD // NHALF
CPW = EH // (NW * CH)
TRASH = N
N_PAD = 10112
RPT = N_PAD // 16
BE = 2048
BN = 2000


def _sc_mesh():
    return plsc.VectorSubcoreMesh(core_axis_name="c", subcore_axis_name="s")


def _gather_rows(nodes, senders_h):
    nbuf = 4

    @functools.partial(
        pl.kernel,
        mesh=_sc_mesh(),
        out_type=jax.ShapeDtypeStruct((EH, D), jnp.float32),
        scratch_types=[
            pltpu.VMEM((CPW * CH,), jnp.int32),
            pltpu.VMEM((nbuf, CH, D), jnp.float32),
            [pltpu.SemaphoreType.DMA] * nbuf,
            [pltpu.SemaphoreType.DMA] * nbuf,
        ],
        compiler_params=pltpu.CompilerParams(use_tc_tiling_on_sc=False),
    )
    def k(nodes_hbm, snd_hbm, out_hbm, idx_v, rows_v, gsems, wsems):
        wid = lax.axis_index("s") * 2 + lax.axis_index("c")
        base = wid * (CPW * CH)
        pltpu.sync_copy(snd_hbm.at[pl.ds(base, CPW * CH)], idx_v)

        def body(j, carry):
            gh = []
            for b in range(nbuf):
                i = j * nbuf + b
                gh.append(pltpu.async_copy(
                    nodes_hbm.at[idx_v.at[pl.ds(i * CH, CH)]],
                    rows_v.at[b], gsems[b]))
            wh = []
            for b in range(nbuf):
                i = j * nbuf + b
                gh[b].wait()
                wh.append(pltpu.async_copy(
                    rows_v.at[b], out_hbm.at[pl.ds(base + i * CH, CH)],
                    wsems[b]))
            for b in range(nbuf):
                wh[b].wait()
            return carry

        lax.fori_loop(0, CPW // nbuf, body, 0)

    return k(nodes, senders_h)


def _edge_matmul(x_src, edges_h, w_cat):
    def body(x_ref, e_ref, w_ref, o_ref):
        x = x_ref[...]
        e = e_ref[...]
        y = jnp.dot(x, w_ref[...], preferred_element_type=jnp.float32)
        acc = jnp.zeros((BE, DPAD), jnp.float32)
        for f in range(DE):
            acc = acc + e[:, f:f + 1] * y[:, f * DPAD:(f + 1) * DPAD]
        col = lax.broadcasted_iota(jnp.int32, (BE, DPAD), 1)
        o_ref[...] = acc + jnp.where(col == D, 1.0, 0.0)

    return pl.pallas_call(
        body,
        grid=(EH // BE,),
        in_specs=[
            pl.BlockSpec((BE, D), lambda i: (i, 0)),
            pl.BlockSpec((BE, DE), lambda i: (i, 0)),
            pl.BlockSpec((D, DE * DPAD), lambda i: (0, 0)),
        ],
        out_specs=pl.BlockSpec((BE, DPAD), lambda i: (i, 0)),
        out_shape=jax.ShapeDtypeStruct((EH, DPAD), jnp.float32),
    )(x_src, edges_h, w_cat)


def _scatter_add(msgs, receivers_h, zinit):
    @functools.partial(
        pl.kernel,
        mesh=_sc_mesh(),
        out_type=jax.ShapeDtypeStruct((2, N_PAD, DPAD), jnp.float32),
        scratch_types=[
            pltpu.VMEM((CH,), jnp.int32),
            pltpu.VMEM((CH, DPAD), jnp.float32),
            pltpu.VMEM_SHARED((N_PAD, DPAD), jnp.float32),
        ],
    )
    def k(msgs_hbm, rcv_hbm, z_hbm, out_hbm, idx_v, m_v, agg_sh):
        c = lax.axis_index("c")
        s = lax.axis_index("s")
        wid = s * 2 + c
        pltpu.sync_copy(z_hbm.at[pl.ds(s * RPT, RPT)], agg_sh.at[pl.ds(s * RPT, RPT)])
        plsc.subcore_barrier()
        base = wid * (CPW * CH)

        def body(i, carry):
            off = base + i * CH
            pltpu.sync_copy(rcv_hbm.at[pl.ds(off, CH)], idx_v)
            pltpu.sync_copy(msgs_hbm.at[pl.ds(off, CH)], m_v)
            pltpu.sync_copy(m_v, agg_sh.at[idx_v], add=True)
            return carry

        lax.fori_loop(0, CPW, body, 0)
        plsc.subcore_barrier()
        pltpu.sync_copy(agg_sh.at[pl.ds(s * RPT, RPT)],
                        out_hbm.at[c, pl.ds(s * RPT, RPT)])

    return k(msgs, receivers_h, zinit)


def _finalize(parts, nodes, w_root, b_root, gn_scale, gn_bias):
    def body(*refs):
        p_refs = refs[:NHALF]
        n_ref, wr_ref, br_ref, gs_ref, gb_ref, o_ref = refs[NHALF:]
        agg = sum(p[0] + p[1] for p in p_refs)
        deg = agg[:, D:D + 1]
        a = agg[:, :D] / jnp.maximum(deg, 1.0)
        h = a + jnp.dot(n_ref[...], wr_ref[...],
                        preferred_element_type=jnp.float32) + br_ref[...]
        mean = h
        cen = h - mean
        var = cen * cen
        y = cen * lax.rsqrt(var + EPS)
        o_ref[...] = y * gs_ref[...] + gb_ref[...]

    return pl.pallas_call(
        body,
        grid=(N // BN,),
        in_specs=[
            pl.BlockSpec((2, BN, DPAD), lambda i: (0, i, 0))
            for _ in range(NHALF)
        ] + [
            pl.BlockSpec((BN, D), lambda i: (i, 0)),
            pl.BlockSpec((D, D), lambda i: (0, 0)),
            pl.BlockSpec((1, D), lambda i: (0, 0)),
            pl.BlockSpec((1, D), lambda i: (0, 0)),
            pl.BlockSpec((1, D), lambda i: (0, 0)),
        ],
        out_specs=pl.BlockSpec((BN, D), lambda i: (i, 0)),
        out_shape=jax.ShapeDtypeStruct((N, D), jnp.float32),
    )(*parts, nodes, w_root, b_root, gn_scale, gn_bias)


def kernel(nodes, edges, W_mlp, W_root, b_root, gn_scale, gn_bias, senders, receivers):
    pad = E_PAD - E
    senders_p = jnp.concatenate([senders, jnp.zeros((pad,), jnp.int32)])
    receivers_p = jnp.concatenate([receivers, jnp.full((pad,), TRASH, jnp.int32)])
    edges_p = jnp.pad(edges, ((0, pad), (0, 0)))
    w_cat = jnp.pad(W_mlp, ((0, 0), (0, 0), (0, DPAD - D))).transpose(1, 0, 2)
    w_cat = w_cat.reshape(D, DE * DPAD)
    zinit = jnp.zeros((N_PAD, DPAD), jnp.float32)

    parts = []
    for h in range(NHALF):
        sl = slice(h * EH, (h + 1) * EH)
        x_h = _gather_rows(nodes, senders_p[sl])
        m_h = _edge_matmul(x_h, edges_p[sl], w_cat)
        parts.append(_scatter_add(m_h, receivers_p[sl], zinit))
    out = _finalize(parts, nodes,
                    W_root, b_root.reshape(1, D),
                    gn_scale.reshape(1, D), gn_bias.reshape(1, D))
    return out
